# baseline (speedup 1.0000x reference)
import functools

import numpy as np
import jax
import jax.numpy as jnp
from jax import lax
from jax.experimental import pallas as pl
from jax.experimental.pallas import tpu as pltpu
from jax.experimental.pallas import tpu_sc as plsc

_MAX_LEN = 4096


def _pe_table(d_model):
    pos = np.arange(0, _MAX_LEN, dtype=np.float32)[:, None]
    mul = np.exp(
        np.arange(0, d_model, 2, dtype=np.float32) * -(np.log(10000.0) / d_model)
    )
    pe = np.zeros((_MAX_LEN, d_model), dtype=np.float32)
    pe[:, 0::2] = np.sin(pos * mul)
    pe[:, 1::2] = np.cos(pos * mul)
    return jnp.asarray(pe)


def kernel(tokens, embed_table):
    B, S = tokens.shape
    V, D = embed_table.shape
    N = B * S
    flat_tok = tokens.reshape(N).astype(jnp.int32)
    pe = _pe_table(D)[:S]

    info = plsc.get_sparse_core_info()
    NC, NS = info.num_cores, info.num_subcores
    NW = NC * NS
    n_per_w = N // NW
    K = 32
    n_chunks = n_per_w // K
    LANES = D // 16

    mesh = plsc.VectorSubcoreMesh(core_axis_name="c", subcore_axis_name="s")

    @functools.partial(
        pl.kernel,
        mesh=mesh,
        out_type=jax.ShapeDtypeStruct((N, D), jnp.float32),
        scratch_types=[
            pltpu.VMEM((n_per_w,), jnp.int32),
            pltpu.VMEM((K, D), jnp.float32),
            pltpu.VMEM((K, D), jnp.float32),
            pltpu.SemaphoreType.DMA,
        ],
    )
    def emb_kernel(tok_hbm, table_hbm, pe_hbm, out_hbm, idx_v, rows_v, pe_v, sem):
        wid = lax.axis_index("s") * NC + lax.axis_index("c")
        base = wid * n_per_w
        s_base = lax.rem(base, S)
        pltpu.sync_copy(tok_hbm.at[pl.ds(base, n_per_w)], idx_v)

        def chunk_body(c, _):
            off = c * K
            gather = pltpu.async_copy(
                table_hbm.at[idx_v.at[pl.ds(off, K)]], rows_v, sem
            )
            pltpu.sync_copy(pe_hbm.at[pl.ds(s_base + off, K)], pe_v)
            gather.wait()

            def add_row(r, _):
                for j in range(LANES):
                    sl = pl.ds(j * 16, 16)
                    rows_v[r, sl] = rows_v[r, sl] + pe_v[r, sl]
                return 0

            lax.fori_loop(0, K, add_row, 0)
            pltpu.sync_copy(rows_v, out_hbm.at[pl.ds(base + off, K)])
            return 0

        lax.fori_loop(0, n_chunks, chunk_body, 0)

    out = emb_kernel(flat_tok, embed_table, pe)
    return out.reshape(B, S, D)
